# baseline (speedup 1.0000x reference)
import functools

import jax
import jax.numpy as jnp
from jax import lax
from jax.experimental import pallas as pl
from jax.experimental.pallas import tpu as pltpu
from jax.experimental.pallas import tpu_sc as plsc

NUM_ENTITIES = 1000000
NUM_RELATIONS = 1000
EMBED_DIM = 64
BATCH = 16384

_INFO = plsc.get_sparse_core_info()
_NC = _INFO.num_cores
_NS = _INFO.num_subcores
_L = _INFO.num_lanes
_NW = _NC * _NS
_BPW = BATCH // _NW
_HALF = _BPW // 2
_GCHUNK = 128


def _sqrt16(x):
    xc = jnp.maximum(x, jnp.float32(1e-35))
    i = plsc.bitcast(xc, jnp.int32)
    y = plsc.bitcast(jnp.int32(0x5F3759DF) - (i >> 1), jnp.float32)
    half = jnp.float32(0.5) * xc
    for _ in range(4):
        y = y * (jnp.float32(1.5) - half * y * y)
    return x * y


def _transe_body(head_hbm, rel_hbm, tail_hbm, ent2, rel2, out_hbm,
                 hidx, ridx, tidx, phidx, pridx, ptidx,
                 hrows, rrows, trows, score_v, sem):
    wid = lax.axis_index("s") * _NC + lax.axis_index("c")
    base = wid * _BPW

    pltpu.sync_copy(head_hbm.at[pl.ds(base, _BPW)], hidx)
    pltpu.sync_copy(rel_hbm.at[pl.ds(base, _BPW)], ridx)
    pltpu.sync_copy(tail_hbm.at[pl.ds(base, _BPW)], tidx)

    for v in range(_BPW // _L):
        sl = pl.ds(v * _L, _L)
        phidx[sl] = hidx[sl] >> 1
        pridx[sl] = ridx[sl] >> 1
        ptidx[sl] = tidx[sl] >> 1

    lane = lax.iota(jnp.int32, _L)
    zero = jnp.zeros((_L,), jnp.float32)

    for half in range(2):
        hbase = half * _HALF
        copies = []
        for j in range(_HALF // _GCHUNK):
            isl = pl.ds(hbase + j * _GCHUNK, _GCHUNK)
            dsl = pl.ds(j * _GCHUNK, _GCHUNK)
            copies.append(pltpu.async_copy(
                ent2.at[phidx.at[isl]], hrows.at[dsl], sem))
            copies.append(pltpu.async_copy(
                rel2.at[pridx.at[isl]], rrows.at[dsl], sem))
            copies.append(pltpu.async_copy(
                ent2.at[ptidx.at[isl]], trows.at[dsl], sem))
        for cp in copies:
            cp.wait()

        def group_body(g, _):
            isl = pl.ds(hbase + g * _L, _L)
            hv = hidx[isl]
            rv = ridx[isl]
            tv = tidx[isl]
            hoff = (hv & 1) * EMBED_DIM
            roff = (rv & 1) * EMBED_DIM
            toff = (tv & 1) * EMBED_DIM
            out_vec = zero
            for l in range(_L):
                row = g * _L + l
                ho = hoff[l]
                ro = roff[l]
                to = toff[l]
                acc = zero
                for c in range(EMBED_DIM // _L):
                    h = hrows[row, pl.ds(ho + c * _L, _L)]
                    r = rrows[row, pl.ds(ro + c * _L, _L)]
                    t = trows[row, pl.ds(to + c * _L, _L)]
                    d = h + r - t
                    acc = acc + d * d
                s = jnp.sum(acc)
                out_vec = jnp.where(lane == l, s, out_vec)
            score_v[pl.ds(hbase + g * _L, _L)] = _sqrt16(out_vec)
            return 0

        lax.fori_loop(0, _HALF // _L, group_body, 0)

    pltpu.sync_copy(score_v, out_hbm.at[pl.ds(base, _BPW)])


@jax.jit
def kernel(head, relation, tail, entity_emb, relation_emb):
    ent2 = entity_emb.reshape(NUM_ENTITIES // 2, 2 * EMBED_DIM)
    rel2 = relation_emb.reshape(NUM_RELATIONS // 2, 2 * EMBED_DIM)
    mesh = plsc.VectorSubcoreMesh(core_axis_name="c", subcore_axis_name="s")
    k = functools.partial(
        pl.kernel,
        mesh=mesh,
        out_type=jax.ShapeDtypeStruct((BATCH,), jnp.float32),
        scratch_types=[
            pltpu.VMEM((_BPW,), jnp.int32),
            pltpu.VMEM((_BPW,), jnp.int32),
            pltpu.VMEM((_BPW,), jnp.int32),
            pltpu.VMEM((_BPW,), jnp.int32),
            pltpu.VMEM((_BPW,), jnp.int32),
            pltpu.VMEM((_BPW,), jnp.int32),
            pltpu.VMEM((_HALF, 2 * EMBED_DIM), jnp.float32),
            pltpu.VMEM((_HALF, 2 * EMBED_DIM), jnp.float32),
            pltpu.VMEM((_HALF, 2 * EMBED_DIM), jnp.float32),
            pltpu.VMEM((_BPW,), jnp.float32),
            pltpu.SemaphoreType.DMA,
        ],
        compiler_params=pltpu.CompilerParams(
            needs_layout_passes=False, use_tc_tiling_on_sc=True,
            skip_device_barrier=True),
    )(_transe_body)
    return k(head, relation, tail, ent2, rel2)


# speedup vs baseline: 1.4746x; 1.4746x over previous
import functools

import jax
import jax.numpy as jnp
from jax import lax
from jax.experimental import pallas as pl
from jax.experimental.pallas import tpu as pltpu
from jax.experimental.pallas import tpu_sc as plsc

NUM_ENTITIES = 1000000
NUM_RELATIONS = 1000
EMBED_DIM = 64
BATCH = 16384

_INFO = plsc.get_sparse_core_info()
_NC = _INFO.num_cores
_NS = _INFO.num_subcores
_L = _INFO.num_lanes
_NW = _NC * _NS
_BPW = BATCH // _NW
_CHUNK = 16
_NCHUNK = _BPW // _CHUNK


def _sqrt16(x):
    xc = jnp.maximum(x, jnp.float32(1e-35))
    i = plsc.bitcast(xc, jnp.int32)
    y = plsc.bitcast(jnp.int32(0x5F3759DF) - (i >> 1), jnp.float32)
    half = jnp.float32(0.5) * xc
    for _ in range(4):
        y = y * (jnp.float32(1.5) - half * y * y)
    return x * y


def _transe_body(head_hbm, rel_hbm, tail_hbm, ent2d, rel2, out_hbm,
                 hidx, ridx, tidx, sridx,
                 hslab, tslab, rrows, score_v, sem):
    wid = lax.axis_index("s") * _NC + lax.axis_index("c")
    base = wid * _BPW

    pltpu.sync_copy(head_hbm.at[pl.ds(base, _BPW)], hidx)
    pltpu.sync_copy(rel_hbm.at[pl.ds(base, _BPW)], ridx)
    pltpu.sync_copy(tail_hbm.at[pl.ds(base, _BPW)], tidx)

    for v in range(_BPW // _L):
        sl = pl.ds(v * _L, _L)
        sridx[sl] = ridx[sl] >> 1

    lane = lax.iota(jnp.int32, _L)
    zero = jnp.zeros((_L,), jnp.float32)

    def chunk_body(c, _):
        isl = pl.ds(c * _CHUNK, _CHUNK)
        hv = hidx[isl]
        tv = tidx[isl]
        rv = ridx[isl]
        hp = (hv >> 3) << 3
        tp = (tv >> 3) << 3
        copies = [pltpu.async_copy(rel2.at[sridx.at[isl]], rrows, sem)]
        for s in range(_CHUNK):
            copies.append(pltpu.async_copy(
                ent2d.at[pl.ds(pl.multiple_of(hp[s], 8), 8), :],
                hslab.at[s], sem))
            copies.append(pltpu.async_copy(
                ent2d.at[pl.ds(pl.multiple_of(tp[s], 8), 8), :],
                tslab.at[s], sem))
        for cp in copies:
            cp.wait()

        hrow = hv & 7
        trow = tv & 7
        roff = (rv & 1) * EMBED_DIM
        out_vec = zero
        for l in range(_L):
            hr = hrow[l]
            tr = trow[l]
            ro = roff[l]
            acc = zero
            for q in range(EMBED_DIM // _L):
                h = hslab[l, hr, pl.ds(q * _L, _L)]
                t = tslab[l, tr, pl.ds(q * _L, _L)]
                r = rrows[l, pl.ds(ro + q * _L, _L)]
                d = h + r - t
                acc = acc + d * d
            s = jnp.sum(acc)
            out_vec = jnp.where(lane == l, s, out_vec)
        score_v[isl] = _sqrt16(out_vec)
        return 0

    lax.fori_loop(0, _NCHUNK, chunk_body, 0)

    pltpu.sync_copy(score_v, out_hbm.at[pl.ds(base, _BPW)])


@jax.jit
def kernel(head, relation, tail, entity_emb, relation_emb):
    rel2 = relation_emb.reshape(NUM_RELATIONS // 2, 2 * EMBED_DIM)
    mesh = plsc.VectorSubcoreMesh(core_axis_name="c", subcore_axis_name="s")
    k = functools.partial(
        pl.kernel,
        mesh=mesh,
        out_type=jax.ShapeDtypeStruct((BATCH,), jnp.float32),
        scratch_types=[
            pltpu.VMEM((_BPW,), jnp.int32),
            pltpu.VMEM((_BPW,), jnp.int32),
            pltpu.VMEM((_BPW,), jnp.int32),
            pltpu.VMEM((_BPW,), jnp.int32),
            pltpu.VMEM((_CHUNK, 8, EMBED_DIM), jnp.float32),
            pltpu.VMEM((_CHUNK, 8, EMBED_DIM), jnp.float32),
            pltpu.VMEM((_CHUNK, 2 * EMBED_DIM), jnp.float32),
            pltpu.VMEM((_BPW,), jnp.float32),
            pltpu.SemaphoreType.DMA,
        ],
        compiler_params=pltpu.CompilerParams(
            needs_layout_passes=False, use_tc_tiling_on_sc=True),
    )(_transe_body)
    return k(head, relation, tail, entity_emb, rel2)


# speedup vs baseline: 2.0703x; 1.4040x over previous
import functools

import jax
import jax.numpy as jnp
from jax import lax
from jax.experimental import pallas as pl
from jax.experimental.pallas import tpu as pltpu
from jax.experimental.pallas import tpu_sc as plsc

NUM_ENTITIES = 1000000
NUM_RELATIONS = 1000
EMBED_DIM = 64
BATCH = 16384

_INFO = plsc.get_sparse_core_info()
_NC = _INFO.num_cores
_NS = _INFO.num_subcores
_L = _INFO.num_lanes
_NW = _NC * _NS
_BPW = BATCH // _NW
_CHUNK = 16
_NCHUNK = _BPW // _CHUNK


def _sqrt16(x):
    xc = jnp.maximum(x, jnp.float32(1e-35))
    i = plsc.bitcast(xc, jnp.int32)
    y = plsc.bitcast(jnp.int32(0x5F3759DF) - (i >> 1), jnp.float32)
    half = jnp.float32(0.5) * xc
    for _ in range(4):
        y = y * (jnp.float32(1.5) - half * y * y)
    return x * y


def _transe_body(head_hbm, rel_hbm, tail_hbm, ent2d, rel2, out_hbm,
                 hidx, ridx, tidx, sridx,
                 hslab, tslab, rrows, score_v, sem):
    wid = lax.axis_index("s") * _NC + lax.axis_index("c")
    base = wid * _BPW

    pltpu.sync_copy(head_hbm.at[pl.ds(base, _BPW)], hidx)
    pltpu.sync_copy(rel_hbm.at[pl.ds(base, _BPW)], ridx)
    pltpu.sync_copy(tail_hbm.at[pl.ds(base, _BPW)], tidx)

    for v in range(_BPW // _L):
        sl = pl.ds(v * _L, _L)
        sridx[sl] = ridx[sl] >> 1

    lane = lax.iota(jnp.int32, _L)
    zero = jnp.zeros((_L,), jnp.float32)

    def chunk_body(c, _):
        isl = pl.ds(c * _CHUNK, _CHUNK)
        hv = hidx[isl]
        tv = tidx[isl]
        rv = ridx[isl]
        hp = hv >> 3
        tp = tv >> 3
        copies = [pltpu.async_copy(rel2.at[sridx.at[isl]], rrows, sem)]
        for s in range(_CHUNK):
            copies.append(pltpu.async_copy(
                ent2d.at[hp[s]], hslab.at[s], sem))
            copies.append(pltpu.async_copy(
                ent2d.at[tp[s]], tslab.at[s], sem))
        for cp in copies:
            cp.wait()

        hrow = hv & 7
        trow = tv & 7
        roff = (rv & 1) * EMBED_DIM
        out_vec = zero
        for l in range(_L):
            hr = hrow[l]
            tr = trow[l]
            ro = roff[l]
            acc = zero
            for q in range(EMBED_DIM // _L):
                h = hslab[l, hr, pl.ds(q * _L, _L)]
                t = tslab[l, tr, pl.ds(q * _L, _L)]
                r = rrows[l, pl.ds(ro + q * _L, _L)]
                d = h + r - t
                acc = acc + d * d
            s = jnp.sum(acc)
            out_vec = jnp.where(lane == l, s, out_vec)
        score_v[isl] = _sqrt16(out_vec)
        return 0

    lax.fori_loop(0, _NCHUNK, chunk_body, 0)

    pltpu.sync_copy(score_v, out_hbm.at[pl.ds(base, _BPW)])


@jax.jit
def kernel(head, relation, tail, entity_emb, relation_emb):
    ent3 = entity_emb.reshape(NUM_ENTITIES // 8, 8, EMBED_DIM)
    rel2 = relation_emb.reshape(NUM_RELATIONS // 2, 2 * EMBED_DIM)
    mesh = plsc.VectorSubcoreMesh(core_axis_name="c", subcore_axis_name="s")
    k = functools.partial(
        pl.kernel,
        mesh=mesh,
        out_type=jax.ShapeDtypeStruct((BATCH,), jnp.float32),
        scratch_types=[
            pltpu.VMEM((_BPW,), jnp.int32),
            pltpu.VMEM((_BPW,), jnp.int32),
            pltpu.VMEM((_BPW,), jnp.int32),
            pltpu.VMEM((_BPW,), jnp.int32),
            pltpu.VMEM((_CHUNK, 8, EMBED_DIM), jnp.float32),
            pltpu.VMEM((_CHUNK, 8, EMBED_DIM), jnp.float32),
            pltpu.VMEM((_CHUNK, 2 * EMBED_DIM), jnp.float32),
            pltpu.VMEM((_BPW,), jnp.float32),
            pltpu.SemaphoreType.DMA,
        ],
        compiler_params=pltpu.CompilerParams(
            needs_layout_passes=False, use_tc_tiling_on_sc=True),
    )(_transe_body)
    return k(head, relation, tail, ent3, rel2)
